# baseline (speedup 1.0000x reference)
import jax
import jax.numpy as jnp
from jax.experimental import pallas as pl
from jax.experimental.pallas import tpu as pltpu


def _gcn_fused_kernel(feat_ref, w_ref, adj_ref, out_ref):
    sup = jnp.dot(
        feat_ref[...], w_ref[...], preferred_element_type=jnp.float32
    )
    acc = jnp.dot(
        adj_ref[...],
        sup,
        preferred_element_type=jnp.float32,
    )
    out_ref[...] = jnp.maximum(acc, 0.0).astype(out_ref.dtype)


def _pick_tile(n, cap):
    t = min(cap, n)
    t -= t % 128
    while n % t != 0:
        t -= 128
    return t


def kernel(features, adj, weight):
    n, in_f = features.shape
    in_f2, out_f = weight.shape
    assert in_f == in_f2
    assert adj.shape == (n, n)
    assert n % 128 == 0

    tile_rows = _pick_tile(n, 1024)

    return pl.pallas_call(
        _gcn_fused_kernel,
        out_shape=jax.ShapeDtypeStruct((n, out_f), features.dtype),
        grid_spec=pltpu.PrefetchScalarGridSpec(
            num_scalar_prefetch=0,
            grid=(n // tile_rows,),
            in_specs=[
                pl.BlockSpec((n, in_f), lambda i: (0, 0)),
                pl.BlockSpec((in_f, out_f), lambda i: (0, 0)),
                pl.BlockSpec((tile_rows, n), lambda i: (i, 0)),
            ],
            out_specs=pl.BlockSpec((tile_rows, out_f), lambda i: (i, 0)),
        ),
        compiler_params=pltpu.CompilerParams(
            dimension_semantics=("parallel",)
        ),
    )(features, weight, adj)


# speedup vs baseline: 1.0290x; 1.0290x over previous
import jax
import jax.numpy as jnp
from jax.experimental import pallas as pl
from jax.experimental.pallas import tpu as pltpu


def _gcn_fused_kernel(feat_ref, w_ref, adj_ref, out_ref):
    sup = jnp.dot(
        feat_ref[...], w_ref[...], preferred_element_type=jnp.float32
    )
    acc = jnp.dot(
        adj_ref[...],
        sup,
        preferred_element_type=jnp.float32,
    )
    out_ref[...] = jnp.maximum(acc, 0.0).astype(out_ref.dtype)


def _pick_tile(n, cap):
    t = min(cap, n)
    t -= t % 128
    while n % t != 0:
        t -= 128
    return t


def kernel(features, adj, weight):
    n, in_f = features.shape
    in_f2, out_f = weight.shape
    assert in_f == in_f2
    assert adj.shape == (n, n)
    assert n % 128 == 0

    tile_rows = _pick_tile(n, 512)

    return pl.pallas_call(
        _gcn_fused_kernel,
        out_shape=jax.ShapeDtypeStruct((n, out_f), features.dtype),
        grid_spec=pltpu.PrefetchScalarGridSpec(
            num_scalar_prefetch=0,
            grid=(n // tile_rows,),
            in_specs=[
                pl.BlockSpec((n, in_f), lambda i: (0, 0)),
                pl.BlockSpec((in_f, out_f), lambda i: (0, 0)),
                pl.BlockSpec((tile_rows, n), lambda i: (i, 0)),
            ],
            out_specs=pl.BlockSpec((tile_rows, out_f), lambda i: (i, 0)),
        ),
        compiler_params=pltpu.CompilerParams(
            dimension_semantics=("parallel",)
        ),
    )(features, weight, adj)
